# Initial kernel scaffold; baseline (speedup 1.0000x reference)
import jax
import jax.numpy as jnp
from jax.experimental import pallas as pl


def kernel(input_ids, token_table, position_table):
    raise NotImplementedError("write your pallas kernel here")


# speedup vs baseline: 1.4731x; 1.4731x over previous
import functools

import jax
import jax.numpy as jnp
from jax import lax
from jax.experimental import pallas as pl
from jax.experimental.pallas import tpu as pltpu
from jax.experimental.pallas import tpu_sc as plsc


def kernel(input_ids, token_table, position_table):
    B, S = input_ids.shape
    V, D = token_table.shape
    N = B * S
    L = 16

    info = plsc.get_sparse_core_info()
    NC, NS = info.num_cores, info.num_subcores
    NW = NC * NS
    SB = S // NW

    ids_flat = input_ids.reshape(N).astype(jnp.int32)
    mesh = plsc.VectorSubcoreMesh(core_axis_name="c", subcore_axis_name="s")

    @functools.partial(
        pl.kernel,
        mesh=mesh,
        out_type=jax.ShapeDtypeStruct((N, D), jnp.float32),
        scratch_types=[
            pltpu.VMEM((B * SB,), jnp.int32),
            pltpu.VMEM((SB, D), jnp.float32),
            pltpu.VMEM((SB, D), jnp.float32),
            pltpu.SemaphoreType.DMA,
        ],
    )
    def emb(ids_hbm, tok_hbm, pos_hbm, out_hbm, idx_v, pos_v, tok_v, sem):
        wid = lax.axis_index("s") * NC + lax.axis_index("c")
        s0 = wid * SB
        pltpu.sync_copy(pos_hbm.at[pl.ds(s0, SB)], pos_v)
        for b in range(B):
            pltpu.sync_copy(
                ids_hbm.at[pl.ds(b * S + s0, SB)], idx_v.at[pl.ds(b * SB, SB)]
            )
        for b in range(B):
            pltpu.async_copy(
                tok_hbm.at[idx_v.at[pl.ds(b * SB, SB)]], tok_v, sem
            ).wait()

            def row_add(r, carry):
                for j in range(D // L):
                    plsc.addupdate(
                        tok_v.at[r, pl.ds(j * L, L)],
                        pos_v[r, pl.ds(j * L, L)],
                    )
                return carry

            lax.fori_loop(0, SB, row_add, 0)
            pltpu.sync_copy(tok_v, out_hbm.at[pl.ds(b * S + s0, SB)])

    out = emb(ids_flat, token_table, position_table)
    return out.reshape(B, S, D)
